# Initial kernel scaffold; baseline (speedup 1.0000x reference)
import jax
import jax.numpy as jnp
from jax.experimental import pallas as pl


def kernel(x, anchors):
    raise NotImplementedError("write your pallas kernel here")


# speedup vs baseline: 1.8649x; 1.8649x over previous
import functools

import jax
import jax.numpy as jnp
from jax import lax
from jax.experimental import pallas as pl
from jax.experimental.pallas import tpu as pltpu
from jax.experimental.pallas import tpu_sc as plsc

DIM = 65
DPAD = 128
K = 1024
ALPHA = 0.1
EPS = 1e-7
BLK_A = 512
BLK_C = 2048

NW = 32
GCH = 512


def _argmax_kernel(x_ref, a_ref, idx_ref, tab_ref):
    x = x_ref[...]
    a_s = a_ref[:, 1:]
    a_t = jnp.sqrt(1.0 + jnp.sum(a_s * a_s, axis=1, keepdims=True))
    a_flip = jnp.concatenate([-a_t, a_s], axis=1)

    inner = jax.lax.dot_general(
        x, a_flip, (((1,), (1,)), ((), ())),
        preferred_element_type=jnp.float32)
    idx_ref[0, 0, :] = jnp.argmax(inner, axis=1).astype(jnp.int32)

    @pl.when(pl.program_id(0) == 0)
    def _():
        pad = jnp.zeros((a_ref.shape[0], DPAD - DIM), jnp.float32)
        tab_ref[...] = jnp.concatenate([a_t, a_s, pad], axis=1)


def _tail_kernel(x_ref, y_ref, out_ref):
    x = x_ref[...]
    y = y_ref[:, :DIM]
    x0y0 = x[:, 0:1] * y[:, 0:1]
    alpha_ = jnp.maximum(2.0 * x0y0 - jnp.sum(x * y, axis=1, keepdims=True),
                         1.0 + EPS)
    am1 = alpha_ * alpha_ - 1.0
    d = jnp.log(alpha_ + jnp.sqrt(jnp.maximum(am1, 0.0)))
    sinh_d = jnp.sqrt(jnp.maximum(am1, EPS))
    c = ALPHA * d / sinh_d
    v = c * (y - alpha_ * x)
    v0 = v[:, 0:1]
    vn2 = jnp.sum(v * v, axis=1, keepdims=True) - 2.0 * v0 * v0
    vn = jnp.sqrt(jnp.maximum(vn2, EPS))
    e = jnp.exp(vn)
    einv = 1.0 / e
    out_ref[...] = (0.5 * (e + einv)) * x + (0.5 * (e - einv) / vn) * v


def _make_sc_gather(b):
    b_per_w = b // NW
    n_chunks = b_per_w // GCH
    mesh = plsc.VectorSubcoreMesh(core_axis_name="c", subcore_axis_name="s")

    @functools.partial(
        pl.kernel, mesh=mesh,
        out_type=jax.ShapeDtypeStruct((b, DPAD), jnp.float32),
        scratch_types=[
            pltpu.VMEM((GCH,), jnp.int32),
            pltpu.VMEM((GCH, DPAD), jnp.float32),
            pltpu.SemaphoreType.DMA,
        ],
    )
    def gather(table_hbm, idx_hbm, out_hbm, idx_v, rows_v, sem):
        wid = lax.axis_index("s") * 2 + lax.axis_index("c")
        base = wid * b_per_w
        for ch in range(n_chunks):
            pltpu.sync_copy(idx_hbm.at[pl.ds(base + ch * GCH, GCH)], idx_v)
            pltpu.async_copy(
                table_hbm.at[idx_v], rows_v, sem
            ).wait()
            pltpu.sync_copy(rows_v, out_hbm.at[pl.ds(base + ch * GCH, GCH)])

    return gather


def kernel(x, anchors):
    b = x.shape[0]
    idx3, table = pl.pallas_call(
        _argmax_kernel,
        grid=(b // BLK_A,),
        in_specs=[
            pl.BlockSpec((BLK_A, DIM), lambda i: (i, 0)),
            pl.BlockSpec((K, DIM), lambda i: (0, 0)),
        ],
        out_specs=[
            pl.BlockSpec((1, 1, BLK_A), lambda i: (i, 0, 0)),
            pl.BlockSpec((K, DPAD), lambda i: (0, 0)),
        ],
        out_shape=[
            jax.ShapeDtypeStruct((b // BLK_A, 1, BLK_A), jnp.int32),
            jax.ShapeDtypeStruct((K, DPAD), jnp.float32),
        ],
    )(x, anchors)
    idx = idx3.reshape(b)

    y_pad = _make_sc_gather(b)(table, idx)

    return pl.pallas_call(
        _tail_kernel,
        grid=(b // BLK_C,),
        in_specs=[
            pl.BlockSpec((BLK_C, DIM), lambda i: (i, 0)),
            pl.BlockSpec((BLK_C, DPAD), lambda i: (i, 0)),
        ],
        out_specs=pl.BlockSpec((BLK_C, DIM), lambda i: (i, 0)),
        out_shape=jax.ShapeDtypeStruct(x.shape, x.dtype),
    )(x, y_pad)


# speedup vs baseline: 2.0146x; 1.0802x over previous
import functools

import jax
import jax.numpy as jnp
from jax import lax
from jax.experimental import pallas as pl
from jax.experimental.pallas import tpu as pltpu
from jax.experimental.pallas import tpu_sc as plsc

DIM = 65
DPAD = 128
K = 1024
ALPHA = 0.1
EPS = 1e-7
BLK_A = 512
BLK_C = 2048

NW = 32
GCH = 128


def _argmax_kernel(x_ref, a_ref, m_ref, idx_ref, tab_ref, aflip_ref):
    @pl.when(pl.program_id(0) == 0)
    def _():
        a_s = a_ref[:, 1:]
        a_t = jnp.sqrt(1.0 + jnp.sum(a_s * a_s, axis=1, keepdims=True))
        pad = jnp.zeros((a_ref.shape[0], DPAD - DIM), jnp.float32)
        tab_ref[...] = jnp.concatenate([a_t, a_s, pad], axis=1)
        aflip_ref[...] = jnp.concatenate([-a_t, a_s], axis=1)

    x = x_ref[...]
    inner = jax.lax.dot_general(
        x, aflip_ref[...], (((1,), (1,)), ((), ())),
        preferred_element_type=jnp.float32)
    m = jnp.max(inner, axis=1, keepdims=True)
    rev = (K - 1) - jax.lax.broadcasted_iota(jnp.int32, inner.shape, 1)
    cand = jnp.where(inner >= m, rev, -1)
    idx = (K - 1) - jnp.max(cand, axis=1, keepdims=True)
    m_ref[...] = m
    idx_ref[...] = idx


def _coeff_kernel(m_ref, p_ref, q_ref):
    alpha_ = jnp.maximum(-m_ref[...], 1.0 + EPS)
    am1 = alpha_ * alpha_ - 1.0
    d = jnp.log(alpha_ + jnp.sqrt(jnp.maximum(am1, 0.0)))
    sinh_d = jnp.sqrt(jnp.maximum(am1, EPS))
    c = ALPHA * d / sinh_d
    vn = jnp.sqrt(jnp.maximum(c * c * am1, EPS))
    e = jnp.exp(vn)
    einv = 1.0 / e
    q = (0.5 * (e - einv) / vn) * c
    p_ref[...] = 0.5 * (e + einv) - q * alpha_
    q_ref[...] = q


def _tail_kernel(x_ref, y_ref, p_ref, q_ref, out_ref):
    out_ref[...] = (p_ref[...] * x_ref[...]
                    + q_ref[...] * y_ref[:, :DIM])


def _make_sc_gather(b):
    b_per_w = b // NW
    n_chunks = b_per_w // GCH
    mesh = plsc.VectorSubcoreMesh(core_axis_name="c", subcore_axis_name="s")

    @functools.partial(
        pl.kernel, mesh=mesh,
        out_type=jax.ShapeDtypeStruct((b, DPAD), jnp.float32),
        scratch_types=[
            pltpu.VMEM((GCH,), jnp.int32),
            pltpu.VMEM((GCH, DPAD), jnp.float32),
            pltpu.SemaphoreType.DMA,
        ],
    )
    def gather(table_hbm, idx_hbm, out_hbm, idx_v, rows_v, sem):
        wid = lax.axis_index("s") * 2 + lax.axis_index("c")
        base = wid * b_per_w
        for ch in range(n_chunks):
            off = base + ch * GCH
            pltpu.sync_copy(idx_hbm.at[pl.ds(off, GCH)], idx_v)
            pltpu.async_copy(
                table_hbm.at[idx_v], rows_v, sem
            ).wait()
            pltpu.sync_copy(rows_v, out_hbm.at[pl.ds(off, GCH)])

    return gather


def kernel(x, anchors):
    b = x.shape[0]
    nb = b // BLK_A
    m2, idx2, table = pl.pallas_call(
        _argmax_kernel,
        grid=(nb,),
        in_specs=[
            pl.BlockSpec((BLK_A, DIM), lambda i: (i, 0)),
            pl.BlockSpec((K, DIM), lambda i: (0, 0)),
        ],
        out_specs=[
            pl.BlockSpec((BLK_A, 1), lambda i: (i, 0)),
            pl.BlockSpec((BLK_A, 1), lambda i: (i, 0)),
            pl.BlockSpec((K, DPAD), lambda i: (0, 0)),
        ],
        out_shape=[
            jax.ShapeDtypeStruct((b, 1), jnp.float32),
            jax.ShapeDtypeStruct((b, 1), jnp.int32),
            jax.ShapeDtypeStruct((K, DPAD), jnp.float32),
        ],
        scratch_shapes=[pltpu.VMEM((K, DIM), jnp.float32)],
    )(x, anchors)
    idx = idx2.reshape(b)

    rows = b // 128
    p2, q2 = pl.pallas_call(
        _coeff_kernel,
        out_shape=[
            jax.ShapeDtypeStruct((rows, 128), jnp.float32),
            jax.ShapeDtypeStruct((rows, 128), jnp.float32),
        ],
    )(m2.reshape(rows, 128))

    y_pad = _make_sc_gather(b)(table, idx)

    return pl.pallas_call(
        _tail_kernel,
        grid=(b // BLK_C,),
        in_specs=[
            pl.BlockSpec((BLK_C, DIM), lambda i: (i, 0)),
            pl.BlockSpec((BLK_C, DPAD), lambda i: (i, 0)),
            pl.BlockSpec((BLK_C, 1), lambda i: (i, 0)),
            pl.BlockSpec((BLK_C, 1), lambda i: (i, 0)),
        ],
        out_specs=pl.BlockSpec((BLK_C, DIM), lambda i: (i, 0)),
        out_shape=jax.ShapeDtypeStruct(x.shape, x.dtype),
    )(x, y_pad, p2.reshape(b, 1), q2.reshape(b, 1))


# speedup vs baseline: 2.7488x; 1.3645x over previous
import functools

import jax
import jax.numpy as jnp
from jax import lax
from jax.experimental import pallas as pl
from jax.experimental.pallas import tpu as pltpu
from jax.experimental.pallas import tpu_sc as plsc

DIM = 65
DPAD = 128
K = 1024
ALPHA = 0.1
EPS = 1e-7
BLK_A = 1024
BLK_C = 2048

NW = 32
GCH = 128


def _phase1_kernel(x_ref, a_ref, idx_ref, p_ref, q_ref, tab_ref, aflip_ref):
    @pl.when(pl.program_id(0) == 0)
    def _():
        a_s = a_ref[:, 1:]
        a_t = jnp.sqrt(1.0 + jnp.sum(a_s * a_s, axis=1, keepdims=True))
        pad = jnp.zeros((a_ref.shape[0], DPAD - DIM), jnp.float32)
        tab_ref[...] = jnp.concatenate([a_t, a_s, pad], axis=1)
        aflip_ref[...] = jnp.concatenate([-a_t, a_s], axis=1)

    x = x_ref[...]
    innerT = jax.lax.dot_general(
        aflip_ref[...], x, (((1,), (1,)), ((), ())),
        preferred_element_type=jnp.float32)
    m = jnp.max(innerT, axis=0, keepdims=True)
    rev = (K - 1) - jax.lax.broadcasted_iota(jnp.int32, innerT.shape, 0)
    cand = jnp.where(innerT >= m, rev, -1)
    idx = (K - 1) - jnp.max(cand, axis=0, keepdims=True)

    alpha_ = jnp.maximum(-m, 1.0 + EPS)
    am1 = alpha_ * alpha_ - 1.0
    d = jnp.log(alpha_ + jnp.sqrt(jnp.maximum(am1, 0.0)))
    sinh_d = jnp.sqrt(jnp.maximum(am1, EPS))
    c = ALPHA * d / sinh_d
    vn = jnp.sqrt(jnp.maximum(c * c * am1, EPS))
    e = jnp.exp(vn)
    einv = 1.0 / e
    q = (0.5 * (e - einv) / vn) * c
    p = 0.5 * (e + einv) - q * alpha_

    idx_ref[...] = idx.reshape(BLK_A // 128, 128)
    p_ref[...] = p
    q_ref[...] = q


def _tail_kernel(x_ref, y_ref, p_ref, q_ref, out_ref):
    xT = jnp.transpose(x_ref[...])
    yT = jnp.transpose(y_ref[...])
    outT = p_ref[...] * xT + q_ref[...] * yT[:DIM, :]
    out_ref[...] = jnp.transpose(outT)


def _make_sc_gather(b):
    b_per_w = b // NW
    n_chunks = b_per_w // GCH
    mesh = plsc.VectorSubcoreMesh(core_axis_name="c", subcore_axis_name="s")

    @functools.partial(
        pl.kernel, mesh=mesh,
        out_type=jax.ShapeDtypeStruct((b, DPAD), jnp.float32),
        scratch_types=[
            pltpu.VMEM((b_per_w,), jnp.int32),
            pltpu.VMEM((GCH, DPAD), jnp.float32),
            pltpu.VMEM((GCH, DPAD), jnp.float32),
            pltpu.SemaphoreType.DMA,
            pltpu.SemaphoreType.DMA,
        ],
    )
    def gather(table_hbm, idx_hbm, out_hbm, idx_v, rows0, rows1, sem0, sem1):
        wid = lax.axis_index("s") * 2 + lax.axis_index("c")
        base = wid * b_per_w
        pltpu.sync_copy(idx_hbm.at[pl.ds(base, b_per_w)], idx_v)
        bufs = (rows0, rows1)
        sems = (sem0, sem1)
        copies = [
            pltpu.async_copy(
                table_hbm.at[idx_v.at[pl.ds(ch * GCH, GCH)]],
                bufs[ch % 2], sems[ch % 2])
            for ch in range(2)
        ]
        for ch in range(n_chunks):
            copies[ch].wait()
            pltpu.sync_copy(bufs[ch % 2],
                            out_hbm.at[pl.ds(base + ch * GCH, GCH)])
            if ch + 2 < n_chunks:
                copies.append(pltpu.async_copy(
                    table_hbm.at[idx_v.at[pl.ds((ch + 2) * GCH, GCH)]],
                    bufs[ch % 2], sems[ch % 2]))

    return gather


def kernel(x, anchors):
    b = x.shape[0]
    nb = b // BLK_A
    rows = b // 128
    rpb = BLK_A // 128
    idx2, p2, q2, table = pl.pallas_call(
        _phase1_kernel,
        grid=(nb,),
        in_specs=[
            pl.BlockSpec((BLK_A, DIM), lambda i: (i, 0)),
            pl.BlockSpec((K, DIM), lambda i: (0, 0)),
        ],
        out_specs=[
            pl.BlockSpec((rpb, 128), lambda i: (i, 0)),
            pl.BlockSpec((1, BLK_A), lambda i: (0, i)),
            pl.BlockSpec((1, BLK_A), lambda i: (0, i)),
            pl.BlockSpec((K, DPAD), lambda i: (0, 0)),
        ],
        out_shape=[
            jax.ShapeDtypeStruct((rows, 128), jnp.int32),
            jax.ShapeDtypeStruct((1, b), jnp.float32),
            jax.ShapeDtypeStruct((1, b), jnp.float32),
            jax.ShapeDtypeStruct((K, DPAD), jnp.float32),
        ],
        scratch_shapes=[pltpu.VMEM((K, DIM), jnp.float32)],
    )(x, anchors)

    y_pad = _make_sc_gather(b)(table, idx2.reshape(b))

    return pl.pallas_call(
        _tail_kernel,
        grid=(b // BLK_C,),
        in_specs=[
            pl.BlockSpec((BLK_C, DIM), lambda i: (i, 0)),
            pl.BlockSpec((BLK_C, DPAD), lambda i: (i, 0)),
            pl.BlockSpec((1, BLK_C), lambda i: (0, i)),
            pl.BlockSpec((1, BLK_C), lambda i: (0, i)),
        ],
        out_specs=pl.BlockSpec((BLK_C, DIM), lambda i: (i, 0)),
        out_shape=jax.ShapeDtypeStruct(x.shape, x.dtype),
    )(x, y_pad, p2, q2)


# speedup vs baseline: 3.0377x; 1.1051x over previous
import functools

import jax
import jax.numpy as jnp
from jax import lax
from jax.experimental import pallas as pl
from jax.experimental.pallas import tpu as pltpu
from jax.experimental.pallas import tpu_sc as plsc

DIM = 65
DPAD = 128
K = 1024
ALPHA = 0.1
EPS = 1e-7
BLK_A = 2048
BLK_C = 2048

NW = 32
GCH = 128


def _phase1_kernel(x_ref, a_ref, idx_ref, p_ref, q_ref,
                   tab_ref, aflip_ref, rev_ref):
    @pl.when(pl.program_id(0) == 0)
    def _():
        a_s = a_ref[:, 1:]
        a_t = jnp.sqrt(1.0 + jnp.sum(a_s * a_s, axis=1, keepdims=True))
        pad = jnp.zeros((a_ref.shape[0], DPAD - DIM), jnp.float32)
        tab_ref[...] = jnp.concatenate([a_t, a_s, pad], axis=1)
        aflip_ref[...] = jnp.concatenate([-a_t, a_s], axis=1)
        rev_ref[...] = (
            (K - 1)
            - jax.lax.broadcasted_iota(jnp.int32, (K, 128), 0)
        ).astype(jnp.float32)

    x = x_ref[...]
    innerT = jax.lax.dot_general(
        aflip_ref[...], x, (((1,), (1,)), ((), ())),
        preferred_element_type=jnp.float32)
    m = jnp.max(innerT, axis=0, keepdims=True)
    rev = jnp.broadcast_to(rev_ref[:, :1], innerT.shape)
    cand = jnp.where(innerT >= m, rev, -1.0)
    idx = (float(K - 1) - jnp.max(cand, axis=0, keepdims=True)
           ).astype(jnp.int32)

    alpha_ = jnp.maximum(-m, 1.0 + EPS)
    am1 = alpha_ * alpha_ - 1.0
    d = jnp.log(alpha_ + jnp.sqrt(jnp.maximum(am1, 0.0)))
    sinh_d = jnp.sqrt(jnp.maximum(am1, EPS))
    c = ALPHA * d / sinh_d
    vn = jnp.sqrt(jnp.maximum(c * c * am1, EPS))
    e = jnp.exp(vn)
    einv = 1.0 / e
    q = (0.5 * (e - einv) / vn) * c
    p = 0.5 * (e + einv) - q * alpha_

    idx_ref[...] = idx.reshape(BLK_A)
    p_ref[...] = p
    q_ref[...] = q


def _tail_kernel(x_ref, y_ref, p_ref, q_ref, out_ref):
    xT = jnp.transpose(x_ref[...])
    yT = jnp.transpose(y_ref[...])
    outT = p_ref[...] * xT + q_ref[...] * yT[:DIM, :]
    out_ref[...] = jnp.transpose(outT)


def _make_sc_gather(b):
    b_per_w = b // NW
    n_chunks = b_per_w // GCH
    mesh = plsc.VectorSubcoreMesh(core_axis_name="c", subcore_axis_name="s")

    nbuf = 4

    @functools.partial(
        pl.kernel, mesh=mesh,
        out_type=jax.ShapeDtypeStruct((b, DPAD), jnp.float32),
        scratch_types=(
            [pltpu.VMEM((b_per_w,), jnp.int32)]
            + [pltpu.VMEM((GCH, DPAD), jnp.float32) for _ in range(nbuf)]
            + [pltpu.SemaphoreType.DMA for _ in range(2 * nbuf)]
        ),
    )
    def gather(table_hbm, idx_hbm, out_hbm, idx_v, *rest):
        bufs = rest[:nbuf]
        gsems = rest[nbuf:2 * nbuf]
        dsems = rest[2 * nbuf:]
        wid = lax.axis_index("s") * 2 + lax.axis_index("c")
        base = wid * b_per_w
        pltpu.sync_copy(idx_hbm.at[pl.ds(base, b_per_w)], idx_v)

        def fire(ch):
            return pltpu.async_copy(
                table_hbm.at[idx_v.at[pl.ds(ch * GCH, GCH)]],
                bufs[ch % nbuf], gsems[ch % nbuf])

        gathers = [fire(ch) for ch in range(nbuf)]
        drains = []
        for ch in range(n_chunks):
            gathers[ch].wait()
            drains.append(pltpu.async_copy(
                bufs[ch % nbuf],
                out_hbm.at[pl.ds(base + ch * GCH, GCH)],
                dsems[ch % nbuf]))
            if ch + nbuf < n_chunks:
                drains[ch].wait()
                gathers.append(fire(ch + nbuf))
        for ch in range(n_chunks - nbuf, n_chunks):
            drains[ch].wait()

    return gather


def kernel(x, anchors):
    b = x.shape[0]
    nb = b // BLK_A
    rows = b // 128
    rpb = BLK_A // 128
    idx2, p2, q2, table = pl.pallas_call(
        _phase1_kernel,
        grid=(nb,),
        in_specs=[
            pl.BlockSpec((BLK_A, DIM), lambda i: (i, 0)),
            pl.BlockSpec((K, DIM), lambda i: (0, 0)),
        ],
        out_specs=[
            pl.BlockSpec((BLK_A,), lambda i: (i,)),
            pl.BlockSpec((1, BLK_A), lambda i: (0, i)),
            pl.BlockSpec((1, BLK_A), lambda i: (0, i)),
            pl.BlockSpec((K, DPAD), lambda i: (0, 0)),
        ],
        out_shape=[
            jax.ShapeDtypeStruct((b,), jnp.int32),
            jax.ShapeDtypeStruct((1, b), jnp.float32),
            jax.ShapeDtypeStruct((1, b), jnp.float32),
            jax.ShapeDtypeStruct((K, DPAD), jnp.float32),
        ],
        scratch_shapes=[
            pltpu.VMEM((K, DIM), jnp.float32),
            pltpu.VMEM((K, 128), jnp.float32),
        ],
    )(x, anchors)

    y_pad = _make_sc_gather(b)(table, idx2)

    return pl.pallas_call(
        _tail_kernel,
        grid=(b // BLK_C,),
        in_specs=[
            pl.BlockSpec((BLK_C, DIM), lambda i: (i, 0)),
            pl.BlockSpec((BLK_C, DPAD), lambda i: (i, 0)),
            pl.BlockSpec((1, BLK_C), lambda i: (0, i)),
            pl.BlockSpec((1, BLK_C), lambda i: (0, i)),
        ],
        out_specs=pl.BlockSpec((BLK_C, DIM), lambda i: (i, 0)),
        out_shape=jax.ShapeDtypeStruct(x.shape, x.dtype),
    )(x, y_pad, p2, q2)
